# Initial kernel scaffold; baseline (speedup 1.0000x reference)
import jax
import jax.numpy as jnp
from jax.experimental import pallas as pl


def kernel(x_user, x_item, edge_index_u2i, edge_index_i2u, W_in_user, b_in_user, W_in_item, b_in_item, Wl_u2i, bl_u2i, Wr_u2i, Wl_i2u, bl_i2u, Wr_i2u, final_W, final_b):
    raise NotImplementedError("write your pallas kernel here")


# speedup vs baseline: 2.7766x; 2.7766x over previous
import jax
import jax.numpy as jnp
from jax import lax
from jax.experimental import pallas as pl
from jax.experimental.pallas import tpu as pltpu
from jax.experimental.pallas import tpu_sc as plsc

N_NODES = 10000
E_TOT = 160000
H = 256
HH = 128
N_TILES = 16

SC_CHUNK = 128
SC_NCHUNK = 80
SC_BLK = 40
E_PAD = N_TILES * SC_NCHUNK * SC_CHUNK
TRASH = N_NODES
N_ACC = N_NODES + 16

ROWS_MAIN = 624

CT_W = 128


def _fill_buf(buf, nrows, ncols, val):
    v = jnp.full((16,), val, jnp.float32)

    def body(i, _):
        for k in range(ncols // 16):
            buf[i, pl.ds(k * 16, 16)] = v
        return ()

    lax.fori_loop(0, nrows, body, ())


def _zero_acc(acc, zsrc, s):
    zrows = zsrc.shape[0]
    full, rem = ROWS_MAIN // zrows, ROWS_MAIN % zrows
    for r in range(full):
        pltpu.sync_copy(zsrc, acc.at[pl.ds(s * ROWS_MAIN + r * zrows, zrows)])
    if rem:
        pltpu.sync_copy(zsrc.at[pl.ds(0, rem)],
                        acc.at[pl.ds(s * ROWS_MAIN + full * zrows, rem)])

    @pl.when(s == N_TILES - 1)
    def _():
        pltpu.sync_copy(zsrc.at[pl.ds(0, 32)], acc.at[pl.ds(N_ACC - 32, 32)])


def _writeback(acc, out_ref, s):
    pltpu.sync_copy(acc.at[pl.ds(s * ROWS_MAIN, ROWS_MAIN)],
                    out_ref.at[pl.ds(s * ROWS_MAIN, ROWS_MAIN)])

    @pl.when(s == N_TILES - 1)
    def _():
        pltpu.sync_copy(acc.at[pl.ds(N_NODES - 16, 16)],
                        out_ref.at[pl.ds(N_NODES - 16, 16)])


def _sc_scatter_kernel(y2_hbm, src_hbm, dst_hbm, tok_hbm, out_hbm,
                       acc, src_v, dst_v, rb0, rb1, sem0, sem1):
    del tok_hbm
    c = lax.axis_index("c")
    s = lax.axis_index("s")

    _fill_buf(rb0, SC_CHUNK, HH, 0.0)
    _zero_acc(acc, rb0, s)
    plsc.subcore_barrier()

    def run_half(h):
        yh = y2_hbm.at[h]
        rbufs = (rb0, rb1)
        sems = (sem0, sem1)

        def issue(j, b):
            pltpu.async_copy(yh.at[src_v.at[j]], rbufs[b], sems[b])

        def drain_scatter(j, b):
            pltpu.make_async_copy(yh.at[src_v.at[j]], rbufs[b], sems[b]).wait()
            pltpu.sync_copy(rbufs[b], acc.at[dst_v.at[j]], add=True)

        for blk in range(SC_NCHUNK // SC_BLK):
            pltpu.sync_copy(src_hbm.at[s, pl.ds(blk * SC_BLK, SC_BLK)], src_v)
            pltpu.sync_copy(dst_hbm.at[s, pl.ds(blk * SC_BLK, SC_BLK)], dst_v)

            issue(0, 0)

            def body(p, _):
                for b in range(2):
                    jj = 2 * p + b

                    @pl.when(jj + 1 < SC_BLK)
                    def _():
                        issue(jj + 1, 1 - b)

                    drain_scatter(jj, b)
                return ()

            lax.fori_loop(0, SC_BLK // 2, body, ())

        plsc.subcore_barrier()
        _writeback(acc, out_hbm.at[h], s)

    @pl.when(c == 0)
    def _():
        run_half(0)

    @pl.when(c == 1)
    def _():
        run_half(1)


def _sc_counts_kernel(dst_hbm, tok_hbm, out_hbm, acc, dst_v, ones_v):
    del tok_hbm
    c = lax.axis_index("c")
    s = lax.axis_index("s")

    _fill_buf(ones_v, SC_CHUNK, CT_W, 0.0)
    _zero_acc(acc, ones_v, s)
    _fill_buf(ones_v, SC_CHUNK, CT_W, 1.0)
    plsc.subcore_barrier()

    def run_core(h):
        pltpu.sync_copy(dst_hbm.at[s, pl.ds(h * SC_BLK, SC_BLK)], dst_v)

        def body(j, _):
            pltpu.sync_copy(ones_v, acc.at[dst_v.at[j]], add=True)
            return ()

        lax.fori_loop(0, SC_BLK, body, ())
        plsc.subcore_barrier()
        _writeback(acc, out_hbm.at[h], s)

    @pl.when(c == 0)
    def _():
        run_core(0)

    @pl.when(c == 1)
    def _():
        run_core(1)


def _sc_scatter(y2, src, dst, tok):
    mesh = plsc.VectorSubcoreMesh(core_axis_name="c", subcore_axis_name="s")
    f = pl.kernel(
        _sc_scatter_kernel,
        out_type=jax.ShapeDtypeStruct((2, N_NODES, HH), jnp.float32),
        mesh=mesh,
        scratch_types=[
            pltpu.VMEM_SHARED((N_ACC, HH), jnp.float32),
            pltpu.VMEM((SC_BLK, SC_CHUNK), jnp.int32),
            pltpu.VMEM((SC_BLK, SC_CHUNK), jnp.int32),
            pltpu.VMEM((SC_CHUNK, HH), jnp.float32),
            pltpu.VMEM((SC_CHUNK, HH), jnp.float32),
            pltpu.SemaphoreType.DMA,
            pltpu.SemaphoreType.DMA,
        ],
    )
    return f(y2, src, dst, tok)


def _sc_counts(dst, tok):
    mesh = plsc.VectorSubcoreMesh(core_axis_name="c", subcore_axis_name="s")
    f = pl.kernel(
        _sc_counts_kernel,
        out_type=jax.ShapeDtypeStruct((2, N_NODES, CT_W), jnp.float32),
        mesh=mesh,
        scratch_types=[
            pltpu.VMEM_SHARED((N_ACC, CT_W), jnp.float32),
            pltpu.VMEM((SC_BLK, SC_CHUNK), jnp.int32),
            pltpu.VMEM((SC_CHUNK, CT_W), jnp.float32),
        ],
    )
    return f(dst, tok)


_BM = 512


def _split2(r, o_ref):
    o_ref[0] = r[:, :HH]
    o_ref[1] = r[:, HH:]


def _cat2(x_ref):
    return jnp.concatenate([x_ref[0], x_ref[1]], axis=1)


def _proj_body(x_ref, w_ref, b_ref, o_ref):
    r = jnp.dot(x_ref[...], w_ref[...], preferred_element_type=jnp.float32)
    _split2(jnp.maximum(r + b_ref[...], 0.0), o_ref)


def _proj(x, w, b):
    n = x.shape[0]
    nm = pl.cdiv(n, _BM)
    return pl.pallas_call(
        _proj_body,
        grid=(nm,),
        in_specs=[
            pl.BlockSpec((_BM, H), lambda m: (m, 0)),
            pl.BlockSpec((H, H), lambda m: (0, 0)),
            pl.BlockSpec((1, H), lambda m: (0, 0)),
        ],
        out_specs=pl.BlockSpec((2, _BM, HH), lambda m: (0, m, 0)),
        out_shape=jax.ShapeDtypeStruct((2, n, HH), jnp.float32),
    )(x, w, b.reshape(1, H))


def _combine_body(s_ref, c_ref, x_ref, wl_ref, bl_ref, wr_ref, o_ref):
    ssum = _cat2(s_ref)
    xd = _cat2(x_ref)
    cnt = jnp.maximum(c_ref[0, :, 0:1] + c_ref[1, :, 0:1], 1.0)
    mean = ssum / cnt
    r = (jnp.dot(mean, wl_ref[...], preferred_element_type=jnp.float32)
         + bl_ref[...]
         + jnp.dot(xd, wr_ref[...], preferred_element_type=jnp.float32))
    _split2(jnp.maximum(r, 0.0), o_ref)


def _combine(s2, cnt2, x2, wl, bl, wr):
    n = s2.shape[1]
    nm = pl.cdiv(n, _BM)
    return pl.pallas_call(
        _combine_body,
        grid=(nm,),
        in_specs=[
            pl.BlockSpec((2, _BM, HH), lambda m: (0, m, 0)),
            pl.BlockSpec((2, _BM, CT_W), lambda m: (0, m, 0)),
            pl.BlockSpec((2, _BM, HH), lambda m: (0, m, 0)),
            pl.BlockSpec((H, H), lambda m: (0, 0)),
            pl.BlockSpec((1, H), lambda m: (0, 0)),
            pl.BlockSpec((H, H), lambda m: (0, 0)),
        ],
        out_specs=pl.BlockSpec((2, _BM, HH), lambda m: (0, m, 0)),
        out_shape=jax.ShapeDtypeStruct((2, n, HH), jnp.float32),
    )(s2, cnt2, x2, wl, bl.reshape(1, H), wr)


def _final_body(x_ref, w_ref, b_ref, o_ref):
    xd = _cat2(x_ref)
    o_ref[...] = jnp.dot(xd, w_ref[...], preferred_element_type=jnp.float32) + b_ref[...]


def _final(x2, w, b):
    n = x2.shape[1]
    nm = pl.cdiv(n, _BM)
    return pl.pallas_call(
        _final_body,
        grid=(nm,),
        in_specs=[
            pl.BlockSpec((2, _BM, HH), lambda m: (0, m, 0)),
            pl.BlockSpec((H, H), lambda m: (0, 0)),
            pl.BlockSpec((1, H), lambda m: (0, 0)),
        ],
        out_specs=pl.BlockSpec((_BM, H), lambda m: (m, 0)),
        out_shape=jax.ShapeDtypeStruct((n, H), jnp.float32),
    )(x2, w, b.reshape(1, H))


def _pad_edges(src, dst):
    pad = E_PAD - E_TOT
    src_p = jnp.concatenate([src, jnp.zeros((pad,), jnp.int32)])
    dst_p = jnp.concatenate([dst, jnp.full((pad,), TRASH, jnp.int32)])
    return (src_p.reshape(N_TILES, SC_NCHUNK, SC_CHUNK),
            dst_p.reshape(N_TILES, SC_NCHUNK, SC_CHUNK))


def kernel(x_user, x_item, edge_index_u2i, edge_index_i2u,
           W_in_user, b_in_user, W_in_item, b_in_item,
           Wl_u2i, bl_u2i, Wr_u2i, Wl_i2u, bl_i2u, Wr_i2u,
           final_W, final_b):
    ei_u2i = edge_index_u2i.astype(jnp.int32)
    ei_i2u = edge_index_i2u.astype(jnp.int32)
    src_u2i, dst_u2i = _pad_edges(ei_u2i[0], ei_u2i[1])
    src_i2u, dst_i2u = _pad_edges(ei_i2u[0], ei_i2u[1])

    u2 = _proj(x_user, W_in_user, b_in_user)
    i2 = _proj(x_item, W_in_item, b_in_item)

    cnt_i = _sc_counts(dst_u2i, u2[0, :8])
    cnt_u = _sc_counts(dst_i2u, cnt_i[0, :8])

    tok = cnt_u[0, :8]
    for l in range(3):
        s_i = _sc_scatter(u2, src_u2i, dst_u2i, tok)
        s_u = _sc_scatter(i2, src_i2u, dst_i2u, s_i[0, :8])
        tok = s_u[0, :8]
        new_i2 = _combine(s_i, cnt_i, i2, Wl_u2i[l], bl_u2i[l], Wr_u2i[l])
        new_u2 = _combine(s_u, cnt_u, u2, Wl_i2u[l], bl_i2u[l], Wr_i2u[l])
        u2, i2 = new_u2, new_i2

    out_u = _final(u2, final_W, final_b)
    out_it = _final(i2, final_W, final_b)
    return out_u, out_it


# speedup vs baseline: 2.9943x; 1.0784x over previous
import jax
import jax.numpy as jnp
from jax import lax
from jax.experimental import pallas as pl
from jax.experimental.pallas import tpu as pltpu
from jax.experimental.pallas import tpu_sc as plsc

N_NODES = 10000
E_TOT = 160000
H = 256
HH = 128
N_TILES = 16

SC_CHUNK = 128
SC_NCHUNK = 80
SC_BLK = 40
E_PAD = N_TILES * SC_NCHUNK * SC_CHUNK
TRASH = N_NODES
N_ACC = N_NODES + 16

ROWS_MAIN = 624

CT_W = 128


def _fill_buf(buf, nrows, ncols, val):
    v = jnp.full((16,), val, jnp.float32)

    def body(i, _):
        for k in range(ncols // 16):
            buf[i, pl.ds(k * 16, 16)] = v
        return ()

    lax.fori_loop(0, nrows, body, ())


def _zero_acc(acc, zsrc, s):
    zrows = zsrc.shape[0]
    full, rem = ROWS_MAIN // zrows, ROWS_MAIN % zrows
    for r in range(full):
        pltpu.sync_copy(zsrc, acc.at[pl.ds(s * ROWS_MAIN + r * zrows, zrows)])
    if rem:
        pltpu.sync_copy(zsrc.at[pl.ds(0, rem)],
                        acc.at[pl.ds(s * ROWS_MAIN + full * zrows, rem)])

    @pl.when(s == N_TILES - 1)
    def _():
        pltpu.sync_copy(zsrc.at[pl.ds(0, 32)], acc.at[pl.ds(N_ACC - 32, 32)])


def _writeback(acc, out_ref, s):
    pltpu.sync_copy(acc.at[pl.ds(s * ROWS_MAIN, ROWS_MAIN)],
                    out_ref.at[pl.ds(s * ROWS_MAIN, ROWS_MAIN)])

    @pl.when(s == N_TILES - 1)
    def _():
        pltpu.sync_copy(acc.at[pl.ds(N_NODES - 16, 16)],
                        out_ref.at[pl.ds(N_NODES - 16, 16)])


def _sc_scatter_kernel(y2_hbm, src_hbm, dst_hbm, tok_hbm, out_hbm,
                       acc, src_v, dst_v, rb0, rb1, sem0, sem1):
    del tok_hbm
    c = lax.axis_index("c")
    s = lax.axis_index("s")

    _fill_buf(rb0, SC_CHUNK, HH, 0.0)
    _zero_acc(acc, rb0, s)
    plsc.subcore_barrier()

    def run_half(h):
        yh = y2_hbm.at[h]
        rbufs = (rb0, rb1)
        sems = (sem0, sem1)

        def issue(j, b):
            pltpu.async_copy(yh.at[src_v.at[j]], rbufs[b], sems[b])

        def drain_scatter(j, b):
            pltpu.make_async_copy(yh.at[src_v.at[j]], rbufs[b], sems[b]).wait()
            pltpu.sync_copy(rbufs[b], acc.at[dst_v.at[j]], add=True)

        for blk in range(SC_NCHUNK // SC_BLK):
            pltpu.sync_copy(src_hbm.at[s, pl.ds(blk * SC_BLK, SC_BLK)], src_v)
            pltpu.sync_copy(dst_hbm.at[s, pl.ds(blk * SC_BLK, SC_BLK)], dst_v)

            issue(0, 0)

            def body(p, _):
                for b in range(2):
                    jj = 2 * p + b

                    @pl.when(jj + 1 < SC_BLK)
                    def _():
                        issue(jj + 1, 1 - b)

                    drain_scatter(jj, b)
                return ()

            lax.fori_loop(0, SC_BLK // 2, body, ())

        plsc.subcore_barrier()
        _writeback(acc, out_hbm.at[h], s)

    @pl.when(c == 0)
    def _():
        run_half(0)

    @pl.when(c == 1)
    def _():
        run_half(1)


def _sc_counts_kernel(dst_hbm, tok_hbm, out_hbm, acc, dst_v, ones_v):
    del tok_hbm
    c = lax.axis_index("c")
    s = lax.axis_index("s")

    _fill_buf(ones_v, SC_CHUNK, CT_W, 0.0)
    _zero_acc(acc, ones_v, s)
    _fill_buf(ones_v, SC_CHUNK, CT_W, 1.0)
    plsc.subcore_barrier()

    def run_core(h):
        pltpu.sync_copy(dst_hbm.at[s, pl.ds(h * SC_BLK, SC_BLK)], dst_v)

        def body(j, _):
            pltpu.sync_copy(ones_v, acc.at[dst_v.at[j]], add=True)
            return ()

        lax.fori_loop(0, SC_BLK, body, ())
        plsc.subcore_barrier()
        _writeback(acc, out_hbm.at[h], s)

    @pl.when(c == 0)
    def _():
        run_core(0)

    @pl.when(c == 1)
    def _():
        run_core(1)


def _sc_scatter(y2, src, dst, tok):
    mesh = plsc.VectorSubcoreMesh(core_axis_name="c", subcore_axis_name="s")
    f = pl.kernel(
        _sc_scatter_kernel,
        out_type=jax.ShapeDtypeStruct((2, N_NODES, HH), jnp.float32),
        mesh=mesh,
        scratch_types=[
            pltpu.VMEM_SHARED((N_ACC, HH), jnp.float32),
            pltpu.VMEM((SC_BLK, SC_CHUNK), jnp.int32),
            pltpu.VMEM((SC_BLK, SC_CHUNK), jnp.int32),
            pltpu.VMEM((SC_CHUNK, HH), jnp.float32),
            pltpu.VMEM((SC_CHUNK, HH), jnp.float32),
            pltpu.SemaphoreType.DMA,
            pltpu.SemaphoreType.DMA,
        ],
    )
    return f(y2, src, dst, tok)


def _sc_counts(dst, tok):
    mesh = plsc.VectorSubcoreMesh(core_axis_name="c", subcore_axis_name="s")
    f = pl.kernel(
        _sc_counts_kernel,
        out_type=jax.ShapeDtypeStruct((2, N_NODES, CT_W), jnp.float32),
        mesh=mesh,
        scratch_types=[
            pltpu.VMEM_SHARED((N_ACC, CT_W), jnp.float32),
            pltpu.VMEM((SC_BLK, SC_CHUNK), jnp.int32),
            pltpu.VMEM((SC_CHUNK, CT_W), jnp.float32),
        ],
    )
    return f(dst, tok)


_BM = 512


def _split2(r, o_ref):
    o_ref[0] = r[:, :HH]
    o_ref[1] = r[:, HH:]


def _cat2(x_ref):
    return jnp.concatenate([x_ref[0], x_ref[1]], axis=1)


def _proj_body(x_ref, w_ref, b_ref, o_ref):
    r = jnp.dot(x_ref[...], w_ref[...], preferred_element_type=jnp.float32)
    _split2(jnp.maximum(r + b_ref[...], 0.0), o_ref)


def _proj(x, w, b):
    n = x.shape[0]
    nm = pl.cdiv(n, _BM)
    return pl.pallas_call(
        _proj_body,
        grid=(nm,),
        in_specs=[
            pl.BlockSpec((_BM, H), lambda m: (m, 0)),
            pl.BlockSpec((H, H), lambda m: (0, 0)),
            pl.BlockSpec((1, H), lambda m: (0, 0)),
        ],
        out_specs=pl.BlockSpec((2, _BM, HH), lambda m: (0, m, 0)),
        out_shape=jax.ShapeDtypeStruct((2, n, HH), jnp.float32),
    )(x, w, b.reshape(1, H))


def _combine_body(s_ref, c_ref, x_ref, wl_ref, bl_ref, wr_ref, o_ref):
    ssum = _cat2(s_ref)
    xd = _cat2(x_ref)
    cnt = jnp.maximum(c_ref[0, :, 0:1] + c_ref[1, :, 0:1], 1.0)
    mean = ssum / cnt
    r = (jnp.dot(mean, wl_ref[...], preferred_element_type=jnp.float32)
         + bl_ref[...]
         + jnp.dot(xd, wr_ref[...], preferred_element_type=jnp.float32))
    _split2(jnp.maximum(r, 0.0), o_ref)


def _combine(s2, cnt2, x2, wl, bl, wr):
    n = s2.shape[1]
    nm = pl.cdiv(n, _BM)
    return pl.pallas_call(
        _combine_body,
        grid=(nm,),
        in_specs=[
            pl.BlockSpec((2, _BM, HH), lambda m: (0, m, 0)),
            pl.BlockSpec((2, _BM, CT_W), lambda m: (0, m, 0)),
            pl.BlockSpec((2, _BM, HH), lambda m: (0, m, 0)),
            pl.BlockSpec((H, H), lambda m: (0, 0)),
            pl.BlockSpec((1, H), lambda m: (0, 0)),
            pl.BlockSpec((H, H), lambda m: (0, 0)),
        ],
        out_specs=pl.BlockSpec((2, _BM, HH), lambda m: (0, m, 0)),
        out_shape=jax.ShapeDtypeStruct((2, n, HH), jnp.float32),
    )(s2, cnt2, x2, wl, bl.reshape(1, H), wr)


def _final_body(x_ref, w_ref, b_ref, o_ref):
    xd = _cat2(x_ref)
    o_ref[...] = jnp.dot(xd, w_ref[...], preferred_element_type=jnp.float32) + b_ref[...]


def _final(x2, w, b):
    n = x2.shape[1]
    nm = pl.cdiv(n, _BM)
    return pl.pallas_call(
        _final_body,
        grid=(nm,),
        in_specs=[
            pl.BlockSpec((2, _BM, HH), lambda m: (0, m, 0)),
            pl.BlockSpec((H, H), lambda m: (0, 0)),
            pl.BlockSpec((1, H), lambda m: (0, 0)),
        ],
        out_specs=pl.BlockSpec((_BM, H), lambda m: (m, 0)),
        out_shape=jax.ShapeDtypeStruct((n, H), jnp.float32),
    )(x2, w, b.reshape(1, H))


def _pad_edges(src, dst):
    pad = E_PAD - E_TOT
    src_p = jnp.concatenate([src, jnp.zeros((pad,), jnp.int32)])
    trash = TRASH + (jnp.arange(pad, dtype=jnp.int32) % 16)
    dst_p = jnp.concatenate([dst, trash])
    return (src_p.reshape(N_TILES, SC_NCHUNK, SC_CHUNK),
            dst_p.reshape(N_TILES, SC_NCHUNK, SC_CHUNK))


def kernel(x_user, x_item, edge_index_u2i, edge_index_i2u,
           W_in_user, b_in_user, W_in_item, b_in_item,
           Wl_u2i, bl_u2i, Wr_u2i, Wl_i2u, bl_i2u, Wr_i2u,
           final_W, final_b):
    ei_u2i = edge_index_u2i.astype(jnp.int32)
    ei_i2u = edge_index_i2u.astype(jnp.int32)
    src_u2i, dst_u2i = _pad_edges(ei_u2i[0], ei_u2i[1])
    src_i2u, dst_i2u = _pad_edges(ei_i2u[0], ei_i2u[1])

    u2 = _proj(x_user, W_in_user, b_in_user)
    i2 = _proj(x_item, W_in_item, b_in_item)

    cnt_i = _sc_counts(dst_u2i, u2[0, :8])
    cnt_u = _sc_counts(dst_i2u, cnt_i[0, :8])

    tok = cnt_u[0, :8]
    for l in range(3):
        s_i = _sc_scatter(u2, src_u2i, dst_u2i, tok)
        s_u = _sc_scatter(i2, src_i2u, dst_i2u, s_i[0, :8])
        tok = s_u[0, :8]
        new_i2 = _combine(s_i, cnt_i, i2, Wl_u2i[l], bl_u2i[l], Wr_u2i[l])
        new_u2 = _combine(s_u, cnt_u, u2, Wl_i2u[l], bl_i2u[l], Wr_i2u[l])
        u2, i2 = new_u2, new_i2

    out_u = _final(u2, final_W, final_b)
    out_it = _final(i2, final_W, final_b)
    return out_u, out_it


# speedup vs baseline: 3.0345x; 1.0134x over previous
import jax
import jax.numpy as jnp
from jax import lax
from jax.experimental import pallas as pl
from jax.experimental.pallas import tpu as pltpu
from jax.experimental.pallas import tpu_sc as plsc

N_NODES = 10000
E_TOT = 160000
H = 256
HH = 128
N_TILES = 16

SC_CHUNK = 128
SC_NCHUNK = 80
SC_BLK = 40
E_PAD = N_TILES * SC_NCHUNK * SC_CHUNK
TRASH = N_NODES
N_ACC = N_NODES + 16

ROWS_MAIN = 624


def _fill_buf(buf, nrows, ncols, val):
    v = jnp.full((16,), val, jnp.float32)

    def body(i, _):
        for k in range(ncols // 16):
            buf[i, pl.ds(k * 16, 16)] = v
        return ()

    lax.fori_loop(0, nrows, body, ())


def _zero_acc(acc, zsrc, s):
    zrows = zsrc.shape[0]
    full, rem = ROWS_MAIN // zrows, ROWS_MAIN % zrows
    for r in range(full):
        pltpu.sync_copy(zsrc, acc.at[pl.ds(s * ROWS_MAIN + r * zrows, zrows)])
    if rem:
        pltpu.sync_copy(zsrc.at[pl.ds(0, rem)],
                        acc.at[pl.ds(s * ROWS_MAIN + full * zrows, rem)])

    @pl.when(s == N_TILES - 1)
    def _():
        pltpu.sync_copy(zsrc.at[pl.ds(0, 32)], acc.at[pl.ds(N_ACC - 32, 32)])


def _writeback(acc, out_ref, s):
    pltpu.sync_copy(acc.at[pl.ds(s * ROWS_MAIN, ROWS_MAIN)],
                    out_ref.at[pl.ds(s * ROWS_MAIN, ROWS_MAIN)])

    @pl.when(s == N_TILES - 1)
    def _():
        pltpu.sync_copy(acc.at[pl.ds(N_NODES - 16, 16)],
                        out_ref.at[pl.ds(N_NODES - 16, 16)])


def _sc_scatter_kernel(y2_hbm, src_hbm, dst_hbm, tok_hbm, out_hbm,
                       acc, src_v, dst_v, rb0, rb1, sem0, sem1, sem2, sem3):
    del tok_hbm
    c = lax.axis_index("c")
    s = lax.axis_index("s")

    _fill_buf(rb0, SC_CHUNK, HH, 0.0)
    _zero_acc(acc, rb0, s)
    plsc.subcore_barrier()

    def run_half(h):
        yh = y2_hbm.at[h]
        rbufs = (rb0, rb1)
        gsems = (sem0, sem1)
        ssems = (sem2, sem3)

        def issue_g(j, b):
            pltpu.async_copy(yh.at[src_v.at[j]], rbufs[b], gsems[b])

        def wait_g(j, b):
            pltpu.make_async_copy(yh.at[src_v.at[j]], rbufs[b], gsems[b]).wait()

        def issue_s(j, b):
            pltpu.async_copy(rbufs[b], acc.at[dst_v.at[j]], ssems[b], add=True)

        def wait_s(j, b):
            pltpu.make_async_copy(rbufs[b], acc.at[dst_v.at[j]], ssems[b]).wait()

        for blk in range(SC_NCHUNK // SC_BLK):
            pltpu.sync_copy(src_hbm.at[s, pl.ds(blk * SC_BLK, SC_BLK)], src_v)
            pltpu.sync_copy(dst_hbm.at[s, pl.ds(blk * SC_BLK, SC_BLK)], dst_v)

            issue_g(0, 0)

            def body(p, _):
                for b in range(2):
                    jj = 2 * p + b

                    @pl.when(jj >= 1)
                    def _():
                        wait_s(jj - 1, 1 - b)

                    @pl.when(jj + 1 < SC_BLK)
                    def _():
                        issue_g(jj + 1, 1 - b)

                    wait_g(jj, b)
                    issue_s(jj, b)
                return ()

            lax.fori_loop(0, SC_BLK // 2, body, ())
            wait_s(SC_BLK - 1, 1)

        plsc.subcore_barrier()
        _writeback(acc, out_hbm.at[h], s)

    @pl.when(c == 0)
    def _():
        run_half(0)

    @pl.when(c == 1)
    def _():
        run_half(1)


CT_FIRE = 8


def _sc_counts_kernel(dst_hbm, tok_hbm, out_hbm, acc, dst_v, ones_v, sem):
    del tok_hbm
    c = lax.axis_index("c")
    s = lax.axis_index("s")

    _fill_buf(ones_v, SC_CHUNK, 128, 0.0)
    _zero_acc(acc, ones_v, s)
    _fill_buf(ones_v, SC_CHUNK, 128, 1.0)
    plsc.subcore_barrier()

    def run_core(h):
        pltpu.sync_copy(dst_hbm.at[s, pl.ds(h * SC_BLK, SC_BLK)], dst_v)

        def fire_drain(q, _):
            for j in range(CT_FIRE):
                pltpu.async_copy(ones_v, acc.at[dst_v.at[q * CT_FIRE + j]],
                                 sem, add=True)
            for j in range(CT_FIRE):
                pltpu.make_async_copy(ones_v, acc.at[dst_v.at[q * CT_FIRE + j]],
                                      sem).wait()
            return ()

        lax.fori_loop(0, SC_BLK // CT_FIRE, fire_drain, ())
        plsc.subcore_barrier()
        _writeback(acc, out_hbm.at[h], s)

    @pl.when(c == 0)
    def _():
        run_core(0)

    @pl.when(c == 1)
    def _():
        run_core(1)


def _sc_scatter(y2, src, dst, tok):
    mesh = plsc.VectorSubcoreMesh(core_axis_name="c", subcore_axis_name="s")
    f = pl.kernel(
        _sc_scatter_kernel,
        out_type=jax.ShapeDtypeStruct((2, N_NODES, HH), jnp.float32),
        mesh=mesh,
        scratch_types=[
            pltpu.VMEM_SHARED((N_ACC, HH), jnp.float32),
            pltpu.VMEM((SC_BLK, SC_CHUNK), jnp.int32),
            pltpu.VMEM((SC_BLK, SC_CHUNK), jnp.int32),
            pltpu.VMEM((SC_CHUNK, HH), jnp.float32),
            pltpu.VMEM((SC_CHUNK, HH), jnp.float32),
            pltpu.SemaphoreType.DMA,
            pltpu.SemaphoreType.DMA,
            pltpu.SemaphoreType.DMA,
            pltpu.SemaphoreType.DMA,
        ],
    )
    return f(y2, src, dst, tok)


def _sc_counts(dst, tok):
    mesh = plsc.VectorSubcoreMesh(core_axis_name="c", subcore_axis_name="s")
    f = pl.kernel(
        _sc_counts_kernel,
        out_type=jax.ShapeDtypeStruct((2, N_NODES, 128), jnp.float32),
        mesh=mesh,
        scratch_types=[
            pltpu.VMEM_SHARED((N_ACC, 128), jnp.float32),
            pltpu.VMEM((SC_BLK, SC_CHUNK), jnp.int32),
            pltpu.VMEM((SC_CHUNK, 128), jnp.float32),
            pltpu.SemaphoreType.DMA,
        ],
    )
    return f(dst, tok)


_BM = 512


def _split2(r, o_ref):
    o_ref[0] = r[:, :HH]
    o_ref[1] = r[:, HH:]


def _cat2(x_ref):
    return jnp.concatenate([x_ref[0], x_ref[1]], axis=1)


def _proj_body(x_ref, w_ref, b_ref, o_ref):
    r = jnp.dot(x_ref[...], w_ref[...], preferred_element_type=jnp.float32)
    _split2(jnp.maximum(r + b_ref[...], 0.0), o_ref)


def _proj(x, w, b):
    n = x.shape[0]
    nm = pl.cdiv(n, _BM)
    return pl.pallas_call(
        _proj_body,
        grid=(nm,),
        in_specs=[
            pl.BlockSpec((_BM, H), lambda m: (m, 0)),
            pl.BlockSpec((H, H), lambda m: (0, 0)),
            pl.BlockSpec((1, H), lambda m: (0, 0)),
        ],
        out_specs=pl.BlockSpec((2, _BM, HH), lambda m: (0, m, 0)),
        out_shape=jax.ShapeDtypeStruct((2, n, HH), jnp.float32),
    )(x, w, b.reshape(1, H))


def _combine_body(s_ref, c_ref, x_ref, wl_ref, bl_ref, wr_ref, o_ref):
    ssum = _cat2(s_ref)
    xd = _cat2(x_ref)
    cnt = jnp.maximum(c_ref[0, :, 0:1] + c_ref[1, :, 0:1], 1.0)
    mean = ssum / cnt
    r = (jnp.dot(mean, wl_ref[...], preferred_element_type=jnp.float32)
         + bl_ref[...]
         + jnp.dot(xd, wr_ref[...], preferred_element_type=jnp.float32))
    _split2(jnp.maximum(r, 0.0), o_ref)


def _combine(s2, cntf, x2, wl, bl, wr):
    n = s2.shape[1]
    nm = pl.cdiv(n, _BM)
    return pl.pallas_call(
        _combine_body,
        grid=(nm,),
        in_specs=[
            pl.BlockSpec((2, _BM, HH), lambda m: (0, m, 0)),
            pl.BlockSpec((2, _BM, 128), lambda m: (0, m, 0)),
            pl.BlockSpec((2, _BM, HH), lambda m: (0, m, 0)),
            pl.BlockSpec((H, H), lambda m: (0, 0)),
            pl.BlockSpec((1, H), lambda m: (0, 0)),
            pl.BlockSpec((H, H), lambda m: (0, 0)),
        ],
        out_specs=pl.BlockSpec((2, _BM, HH), lambda m: (0, m, 0)),
        out_shape=jax.ShapeDtypeStruct((2, n, HH), jnp.float32),
    )(s2, cntf, x2, wl, bl.reshape(1, H), wr)


def _final_body(x_ref, w_ref, b_ref, o_ref):
    xd = _cat2(x_ref)
    o_ref[...] = jnp.dot(xd, w_ref[...], preferred_element_type=jnp.float32) + b_ref[...]


def _final(x2, w, b):
    n = x2.shape[1]
    nm = pl.cdiv(n, _BM)
    return pl.pallas_call(
        _final_body,
        grid=(nm,),
        in_specs=[
            pl.BlockSpec((2, _BM, HH), lambda m: (0, m, 0)),
            pl.BlockSpec((H, H), lambda m: (0, 0)),
            pl.BlockSpec((1, H), lambda m: (0, 0)),
        ],
        out_specs=pl.BlockSpec((_BM, H), lambda m: (m, 0)),
        out_shape=jax.ShapeDtypeStruct((n, H), jnp.float32),
    )(x2, w, b.reshape(1, H))


def _pad_edges(src, dst):
    pad = E_PAD - E_TOT
    src_p = jnp.concatenate([src, jnp.zeros((pad,), jnp.int32)])
    trash = TRASH + (jnp.arange(pad, dtype=jnp.int32) % 16)
    dst_p = jnp.concatenate([dst, trash])
    return (src_p.reshape(N_TILES, SC_NCHUNK, SC_CHUNK),
            dst_p.reshape(N_TILES, SC_NCHUNK, SC_CHUNK))


def kernel(x_user, x_item, edge_index_u2i, edge_index_i2u,
           W_in_user, b_in_user, W_in_item, b_in_item,
           Wl_u2i, bl_u2i, Wr_u2i, Wl_i2u, bl_i2u, Wr_i2u,
           final_W, final_b):
    ei_u2i = edge_index_u2i.astype(jnp.int32)
    ei_i2u = edge_index_i2u.astype(jnp.int32)
    src_u2i, dst_u2i = _pad_edges(ei_u2i[0], ei_u2i[1])
    src_i2u, dst_i2u = _pad_edges(ei_i2u[0], ei_i2u[1])

    u2 = _proj(x_user, W_in_user, b_in_user)
    i2 = _proj(x_item, W_in_item, b_in_item)

    cnt_i = _sc_counts(dst_u2i, u2[0, :8])
    cnt_u = _sc_counts(dst_i2u, cnt_i[0, :8])

    tok = cnt_u[0, :8]
    for l in range(3):
        s_i = _sc_scatter(u2, src_u2i, dst_u2i, tok)
        s_u = _sc_scatter(i2, src_i2u, dst_i2u, s_i[0, :8])
        tok = s_u[0, :8]
        new_i2 = _combine(s_i, cnt_i, i2, Wl_u2i[l], bl_u2i[l], Wr_u2i[l])
        new_u2 = _combine(s_u, cnt_u, u2, Wl_i2u[l], bl_i2u[l], Wr_i2u[l])
        u2, i2 = new_u2, new_i2

    out_u = _final(u2, final_W, final_b)
    out_it = _final(i2, final_W, final_b)
    return out_u, out_it
